# speedup vs baseline: 1.1161x; 1.1161x over previous
import functools

import jax
import jax.numpy as jnp
from jax.experimental import pallas as pl
from jax.experimental.pallas import tpu as pltpu

_P2 = 49
_W2 = 1024
_QK = 96
_DIM = 96
_HEADS = 8
_CH = _QK // _HEADS
_TOPK = 4
_NWIN = 7
_RH = 32
_SCALE = _QK ** (-0.5)


def _qkv_kernel(x_ref, w_ref, b_ref, q_ref, kv_ref, qs_ref, ks_ref):
    xb = x_ref[0]
    qkv = jnp.dot(xb, w_ref[...], preferred_element_type=jnp.float32)
    qkv = qkv + b_ref[...]
    q = qkv[:, :_QK]
    kv = qkv[:, _QK:]
    q_ref[0] = q
    kv_ref[0] = kv
    qs_ref[0, 0] = jnp.sum(q, axis=0)
    ks_ref[0, 0] = jnp.sum(kv[:, :_QK], axis=0)


def _route_kernel(qs_ref, ks_ref, idx_ref):
    qw = qs_ref[:, 0, :] * (_SCALE / float(_W2))
    kw = ks_ref[:, 0, :] * (1.0 / float(_W2))
    logits = jnp.dot(qw, kw.T, preferred_element_type=jnp.float32)
    col = jax.lax.broadcasted_iota(jnp.int32, (_P2, _P2), 1)
    picks = []
    for _ in range(_TOPK):
        mx = jnp.max(logits, axis=1, keepdims=True)
        is_max = logits == mx
        idx = jnp.min(jnp.where(is_max, col, _P2), axis=1)
        picks.append(idx)
        logits = jnp.where(col == idx[:, None], -jnp.inf, logits)
    idx_ref[...] = jnp.stack(picks, axis=1)


def _attn_kernel(idx_sref, q_ref, kv_ref, o_ref, m_scr, l_scr, acc_scr):
    k_step = pl.program_id(1)

    @pl.when(k_step == 0)
    def _init():
        m_scr[...] = jnp.full((_W2, _HEADS), -jnp.inf, jnp.float32)
        l_scr[...] = jnp.zeros((_W2, _HEADS), jnp.float32)
        acc_scr[...] = jnp.zeros((_W2, _QK), jnp.float32)

    q = q_ref[0]
    kv = kv_ref[0]
    for h in range(_HEADS):
        sl = slice(h * _CH, (h + 1) * _CH)
        qh = q[:, sl] * _SCALE
        kh = kv[:, sl]
        vh = kv[:, _QK + h * _CH:_QK + (h + 1) * _CH]
        s = jnp.dot(qh, kh.T, preferred_element_type=jnp.float32)
        m_prev = m_scr[:, h]
        m_new = jnp.maximum(m_prev, jnp.max(s, axis=1))
        p = jnp.exp(s - m_new[:, None])
        alpha = jnp.exp(m_prev - m_new)
        l_scr[:, h] = alpha * l_scr[:, h] + jnp.sum(p, axis=1)
        m_scr[:, h] = m_new
        pv = jnp.dot(p, vh, preferred_element_type=jnp.float32)
        acc_scr[:, sl] = alpha[:, None] * acc_scr[:, sl] + pv

    @pl.when(k_step == _TOPK - 1)
    def _fin():
        for h in range(_HEADS):
            sl = slice(h * _CH, (h + 1) * _CH)
            o_ref[0, :, sl] = acc_scr[:, sl] / l_scr[:, h][:, None]


def _lepe_out_kernel(vh_ref, a_ref, wo_ref, bo_ref, lw_ref, lb_ref, o_ref):
    v = vh_ref[0]
    acc = jnp.broadcast_to(lb_ref[0], (_RH, _NWIN * _RH, _DIM)).astype(jnp.float32)
    for dy in range(3):
        for dx in range(3):
            acc = acc + v[dy:dy + _RH, dx:dx + _NWIN * _RH, :] * lw_ref[dy, dx]
    a = a_ref[...] + acc
    a2 = a.reshape(_RH * _NWIN * _RH, _DIM)
    o = jnp.dot(a2, wo_ref[...], preferred_element_type=jnp.float32) + bo_ref[...]
    o_ref[...] = o.reshape(_RH, _NWIN * _RH, _DIM)


def kernel(x, W_qkv, b_qkv, W_o, b_o, lepe_w, lepe_b):
    b, Hh, Ww, c = x.shape
    xp = x.reshape(_NWIN, _RH, _NWIN, _RH, c).transpose(0, 2, 1, 3, 4)
    xp = xp.reshape(_P2, _W2, c)

    q, kv, qsum, ksum = pl.pallas_call(
        _qkv_kernel,
        grid=(_P2,),
        in_specs=[
            pl.BlockSpec((1, _W2, c), lambda r: (r, 0, 0)),
            pl.BlockSpec((c, 2 * _QK + _DIM), lambda r: (0, 0)),
            pl.BlockSpec((1, 2 * _QK + _DIM), lambda r: (0, 0)),
        ],
        out_specs=[
            pl.BlockSpec((1, _W2, _QK), lambda r: (r, 0, 0)),
            pl.BlockSpec((1, _W2, _QK + _DIM), lambda r: (r, 0, 0)),
            pl.BlockSpec((1, 1, _QK), lambda r: (r, 0, 0)),
            pl.BlockSpec((1, 1, _QK), lambda r: (r, 0, 0)),
        ],
        out_shape=[
            jax.ShapeDtypeStruct((_P2, _W2, _QK), jnp.float32),
            jax.ShapeDtypeStruct((_P2, _W2, _QK + _DIM), jnp.float32),
            jax.ShapeDtypeStruct((_P2, 1, _QK), jnp.float32),
            jax.ShapeDtypeStruct((_P2, 1, _QK), jnp.float32),
        ],
    )(xp, W_qkv, b_qkv.reshape(1, -1))

    topk_idx = pl.pallas_call(
        _route_kernel,
        out_shape=jax.ShapeDtypeStruct((_P2, _TOPK), jnp.int32),
    )(qsum, ksum)

    attn_out = pl.pallas_call(
        _attn_kernel,
        grid_spec=pltpu.PrefetchScalarGridSpec(
            num_scalar_prefetch=1,
            grid=(_P2, _TOPK),
            in_specs=[
                pl.BlockSpec((1, _W2, _QK), lambda r, k, idx: (r, 0, 0)),
                pl.BlockSpec((1, _W2, _QK + _DIM), lambda r, k, idx: (idx[r, k], 0, 0)),
            ],
            out_specs=pl.BlockSpec((1, _W2, _QK), lambda r, k, idx: (r, 0, 0)),
            scratch_shapes=[
                pltpu.VMEM((_W2, _HEADS), jnp.float32),
                pltpu.VMEM((_W2, _HEADS), jnp.float32),
                pltpu.VMEM((_W2, _QK), jnp.float32),
            ],
        ),
        out_shape=jax.ShapeDtypeStruct((_P2, _W2, _QK), jnp.float32),
    )(topk_idx, q, kv)

    def to_img(t):
        return (t.reshape(_NWIN, _NWIN, _RH, _RH, _DIM)
                 .transpose(0, 2, 1, 3, 4)
                 .reshape(Hh, Ww, _DIM))

    attn_img = to_img(attn_out)
    v_img = to_img(kv[:, :, _QK:])
    vpad = jnp.pad(v_img, ((1, 1), (1, 1), (0, 0)))
    vh = jnp.stack([vpad[_RH * bb:_RH * bb + _RH + 2] for bb in range(_NWIN)])
    w_hwc = lepe_w[:, 0].transpose(1, 2, 0)

    out_img = pl.pallas_call(
        _lepe_out_kernel,
        grid=(_NWIN,),
        in_specs=[
            pl.BlockSpec((1, _RH + 2, Ww + 2, _DIM), lambda r: (r, 0, 0, 0)),
            pl.BlockSpec((_RH, Ww, _DIM), lambda r: (r, 0, 0)),
            pl.BlockSpec((_DIM, _DIM), lambda r: (0, 0)),
            pl.BlockSpec((1, _DIM), lambda r: (0, 0)),
            pl.BlockSpec((3, 3, _DIM), lambda r: (0, 0, 0)),
            pl.BlockSpec((1, _DIM), lambda r: (0, 0)),
        ],
        out_specs=pl.BlockSpec((_RH, Ww, _DIM), lambda r: (r, 0, 0)),
        out_shape=jax.ShapeDtypeStruct((Hh, Ww, _DIM), jnp.float32),
    )(vh, attn_img, W_o, b_o.reshape(1, -1), w_hwc, lepe_b.reshape(1, -1))

    return out_img[None]
